# baseline (speedup 1.0000x reference)
import functools

import jax
import jax.numpy as jnp
from jax import lax
from jax.experimental import pallas as pl
from jax.experimental.pallas import tpu as pltpu
from jax.experimental.pallas import tpu_sc as plsc

_BS = 2048
_C = 32


def _sc_sweep(x, pos_emb):
    batch, seq_len, emb = x.shape
    info = plsc.get_sparse_core_info()
    nw = info.num_cores * info.num_subcores
    rows_per_w = seq_len // nw
    n_chunks = rows_per_w // _C

    @functools.partial(
        pl.kernel,
        mesh=plsc.VectorSubcoreMesh(core_axis_name="c", subcore_axis_name="s"),
        out_type=jax.ShapeDtypeStruct((16,), jnp.float32),
        scratch_types=[
            pltpu.VMEM((_C, emb), jnp.float32),
            pltpu.VMEM((_C, emb), jnp.float32),
            pltpu.VMEM((16,), jnp.float32),
            pltpu.VMEM((16,), jnp.float32),
            pltpu.SemaphoreType.DMA,
            pltpu.SemaphoreType.DMA,
        ],
    )
    def body(x_hbm, pos_hbm, out_hbm, buf0, buf1, xv, pv, sem0, sem1):
        wid = lax.axis_index("s") * info.num_cores + lax.axis_index("c")
        base = wid * rows_per_w
        bufs = [buf0, buf1]
        sems = [sem0, sem1]
        cps = [None, None]
        k = 0
        for b in range(batch):
            for ci in range(n_chunks):
                slot = k % 2
                if cps[slot] is not None:
                    cps[slot].wait()
                row0 = base + ci * _C
                cps[slot] = pltpu.async_copy(
                    x_hbm.at[b, pl.ds(row0, _C)], bufs[slot], sems[slot]
                )
                k += 1
        for cp in cps:
            cp.wait()

        @pl.when(wid == 0)
        def _():
            pltpu.sync_copy(x_hbm.at[0, 0, pl.ds(0, 16)], xv)
            pltpu.sync_copy(pos_hbm.at[0, pl.ds(0, 16)], pv)
            xv[...] = xv[...] + pv[...]
            pltpu.sync_copy(xv, out_hbm)

    return body(x, pos_emb)


def _tc_add_kernel(x_ref, pos_ref, o_ref):
    o_ref[...] = x_ref[...] + pos_ref[...]


def _tc_add(x, pos_emb):
    batch, seq_len, emb = x.shape
    grid = (seq_len // _BS, batch)
    return pl.pallas_call(
        _tc_add_kernel,
        grid=grid,
        in_specs=[
            pl.BlockSpec((1, _BS, emb), lambda s, b: (b, s, 0)),
            pl.BlockSpec((_BS, emb), lambda s, b: (s, 0)),
        ],
        out_specs=pl.BlockSpec((1, _BS, emb), lambda s, b: (b, s, 0)),
        out_shape=jax.ShapeDtypeStruct(x.shape, x.dtype),
    )(x, pos_emb)


def kernel(x, pos_emb):
    out_tc = _tc_add(x, pos_emb)
    out_sc = _sc_sweep(x, pos_emb)
    return lax.dynamic_update_slice(out_tc, out_sc.reshape(1, 1, 16), (0, 0, 0))


# speedup vs baseline: 1.2867x; 1.2867x over previous
import functools

import jax
import jax.numpy as jnp
from jax import lax
from jax.experimental import pallas as pl
from jax.experimental.pallas import tpu as pltpu
from jax.experimental.pallas import tpu_sc as plsc

_BS = 2048
_C = 32


def _sc_block(x, pos_emb, b_sel, row_lo, n_rows):
    batch, seq_len, emb = x.shape
    info = plsc.get_sparse_core_info()
    nw = info.num_cores * info.num_subcores
    rows_per_w = n_rows // nw
    n_chunks = rows_per_w // _C
    vregs_per_row = emb // 16

    @functools.partial(
        pl.kernel,
        mesh=plsc.VectorSubcoreMesh(core_axis_name="c", subcore_axis_name="s"),
        out_type=jax.ShapeDtypeStruct((1, n_rows, emb), jnp.float32),
        scratch_types=[
            pltpu.VMEM((_C, emb), jnp.float32),
            pltpu.VMEM((_C, emb), jnp.float32),
        ],
    )
    def body(x_hbm, pos_hbm, out_hbm, p_v, x_v):
        wid = lax.axis_index("s") * info.num_cores + lax.axis_index("c")
        base = wid * rows_per_w

        def chunk_body(ci, _):
            off = base + ci * _C
            pltpu.sync_copy(pos_hbm.at[pl.ds(row_lo + off, _C)], p_v)
            pltpu.sync_copy(x_hbm.at[b_sel, pl.ds(row_lo + off, _C)], x_v)

            def row_body(r, _):
                def vec_body(j, _):
                    for u in range(4):
                        sl = pl.ds(j * 64 + u * 16, 16)
                        x_v[r, sl] = x_v[r, sl] + p_v[r, sl]
                    return 0

                lax.fori_loop(0, vregs_per_row // 4, vec_body, 0)
                return 0

            lax.fori_loop(0, _C, row_body, 0)
            pltpu.sync_copy(x_v, out_hbm.at[0, pl.ds(off, _C)])
            return 0

        lax.fori_loop(0, n_chunks, chunk_body, 0)

    return body(x, pos_emb)


def _tc_add_kernel(x_ref, pos_ref, o_ref):
    o_ref[...] = x_ref[...] + pos_ref[...]


def _tc_add_15(x, pos_emb):
    batch, seq_len, emb = x.shape
    n_steps = (seq_len // _BS) * batch - 1
    return pl.pallas_call(
        _tc_add_kernel,
        grid=(n_steps,),
        in_specs=[
            pl.BlockSpec((1, _BS, emb), lambda i: (i % 4, i // 4, 0)),
            pl.BlockSpec((_BS, emb), lambda i: (i // 4, 0)),
        ],
        out_specs=pl.BlockSpec((1, _BS, emb), lambda i: (i % 4, i // 4, 0)),
        out_shape=jax.ShapeDtypeStruct(x.shape, x.dtype),
    )(x, pos_emb)


def kernel(x, pos_emb):
    batch, seq_len, emb = x.shape
    out_sc = _sc_block(x, pos_emb, batch - 1, seq_len - _BS, _BS)
    out_tc = _tc_add_15(x, pos_emb)
    return lax.dynamic_update_slice(out_tc, out_sc, (batch - 1, seq_len - _BS, 0))


# speedup vs baseline: 1.6618x; 1.2915x over previous
import jax
import jax.numpy as jnp
from jax.experimental import pallas as pl


_BS = 2048


def _add_kernel(x_ref, pos_ref, o_ref):
    o_ref[...] = x_ref[...] + pos_ref[...]


def kernel(x, pos_emb):
    batch, seq_len, emb = x.shape
    grid = (seq_len // _BS, batch)
    return pl.pallas_call(
        _add_kernel,
        grid=grid,
        in_specs=[
            pl.BlockSpec((1, _BS, emb), lambda s, b: (b, s, 0)),
            pl.BlockSpec((_BS, emb), lambda s, b: (s, 0)),
        ],
        out_specs=pl.BlockSpec((1, _BS, emb), lambda s, b: (b, s, 0)),
        out_shape=jax.ShapeDtypeStruct(x.shape, x.dtype),
    )(x, pos_emb)
